# baseline (speedup 1.0000x reference)
import jax
import jax.numpy as jnp
from jax import lax
from jax.experimental import pallas as pl
from jax.experimental.pallas import tpu as pltpu

N = 50000
C = 128
K = 4
E = K + 1
B = 5000
NB = N // B


def _routing_kernel(xo_ref, xc_ref, cen_ref, enc_ref):
    xo = xo_ref[...]
    s0 = xo & 15
    s1 = xo >> 4
    cnt = [jnp.sum((s == j).astype(jnp.float32)) for s in (s0, s1)
           for j in range(16)]
    tot = cnt[0]
    for c in cnt[1:]:
        tot = tot + c
    xh = [c / tot for c in cnt]
    d2 = []
    for k in range(K):
        acc = (cen_ref[k, 0] - xh[0]) ** 2
        for j in range(1, 32):
            acc = acc + (cen_ref[k, j] - xh[j]) ** 2
        d2.append(acc)
    idx = jnp.int32(0)
    best = d2[0]
    for k in range(1, K):
        pred = d2[k] < best
        idx = jnp.where(pred, jnp.int32(k), idx)
        best = jnp.where(pred, d2[k], best)
    xc = xc_ref[...]
    lane = lax.broadcasted_iota(jnp.int32, xc.shape, 1)
    mc = jnp.max(jnp.where(lane % 4 != 0, xc, 0))
    enc_ref[0, 0] = jnp.where(mc <= 63, jnp.int32(K), idx)


def _forward_kernel(e_ref, xo_ref, fp_ref, bw_ref, lw1_ref, lb1_ref, lw2_ref,
                    lb2_ref, rw1_ref, rw2_ref, pw1_ref, pb1_ref, pw2_ref,
                    pb2_ref, pe_ref, out_ref, bits_ref):
    del e_ref
    i = pl.program_id(0)
    xo = xo_ref[0, :, :]
    iota16 = lax.broadcasted_iota(jnp.int32, (B, 16), 1)
    oh0 = ((xo & 15) == iota16).astype(jnp.float32)
    oh1 = ((xo >> 4) == iota16).astype(jnp.float32)

    def mm(a, b):
        return jnp.dot(a, b, preferred_element_type=jnp.float32)

    h = jnp.maximum(lw1_ref[0] + lb1_ref[0], 0.0)
    row = mm(h, lw2_ref[0]) + lb2_ref[0]

    bw = bw_ref[0]
    bmx = jnp.max(bw, axis=0, keepdims=True)
    be = jnp.exp(bw - bmx)
    wsm = be / jnp.sum(be, axis=0, keepdims=True)
    f = wsm[0:1, :] * row + wsm[1:2, :] * fp_ref[...]

    def resnet(f, j):
        t = jnp.maximum(mm(f, rw1_ref[0, j]), 0.0)
        return f + mm(t, rw2_ref[0, j])

    def head(f, t, oh):
        hh = jnp.maximum(mm(f, pw1_ref[0, t]) + pb1_ref[0, t], 0.0)
        lg = mm(hh, pw2_ref[0, t]) + pb2_ref[0, t]
        mx = jnp.max(lg, axis=1, keepdims=True)
        ex = jnp.exp(lg - mx)
        z = jnp.sum(ex, axis=1, keepdims=True)
        exs = jnp.sum(ex * oh, axis=1, keepdims=True)
        ps = exs / z
        bits = jnp.sum(jnp.clip(-jnp.log2(ps + 1e-10), 0.0, 50.0))
        return bits, f + mm(oh, pe_ref[0, t])

    f = resnet(f, 0)
    f = resnet(f, 1)
    bits_a, f = head(f, 0, oh0)
    f = resnet(f, 2)
    f = resnet(f, 3)
    bits_b, f = head(f, 1, oh1)
    f = resnet(f, 4)
    f = resnet(f, 5)
    out_ref[...] = f
    bits_ref[0, 0, 0] = bits_a + bits_b


def kernel(x_C, x_O, feats_prop, centers, params):
    xo2 = x_O.reshape(400, 125)
    xc2 = x_C.reshape(1000, 200)
    enc = pl.pallas_call(
        _routing_kernel,
        out_shape=jax.ShapeDtypeStruct((1, 1), jnp.int32),
        in_specs=[
            pl.BlockSpec(memory_space=pltpu.VMEM),
            pl.BlockSpec(memory_space=pltpu.VMEM),
            pl.BlockSpec(memory_space=pltpu.SMEM),
        ],
        out_specs=pl.BlockSpec(memory_space=pltpu.SMEM),
    )(xo2, xc2, centers)
    enc1 = enc.reshape((1,))

    p = params
    lb1 = p['local_b1'][:, None, :]
    lb2 = p['local_b2'][:, None, :]

    grid_spec = pltpu.PrefetchScalarGridSpec(
        num_scalar_prefetch=1,
        grid=(NB,),
        in_specs=[
            pl.BlockSpec((1, B, 1), lambda i, e: (i, 0, 0)),
            pl.BlockSpec((B, C), lambda i, e: (i, 0)),
            pl.BlockSpec((1, 2, C), lambda i, e: (e[0], 0, 0)),
            pl.BlockSpec((1, 1, C), lambda i, e: (e[0], 0, 0)),
            pl.BlockSpec((1, 1, C), lambda i, e: (e[0], 0, 0)),
            pl.BlockSpec((1, C, C), lambda i, e: (e[0], 0, 0)),
            pl.BlockSpec((1, 1, C), lambda i, e: (e[0], 0, 0)),
            pl.BlockSpec((1, 6, C, C), lambda i, e: (e[0], 0, 0, 0)),
            pl.BlockSpec((1, 6, C, C), lambda i, e: (e[0], 0, 0, 0)),
            pl.BlockSpec((1, 2, C, C), lambda i, e: (e[0], 0, 0, 0)),
            pl.BlockSpec((1, 2, C), lambda i, e: (e[0], 0, 0)),
            pl.BlockSpec((1, 2, C, 16), lambda i, e: (e[0], 0, 0, 0)),
            pl.BlockSpec((1, 2, 16), lambda i, e: (e[0], 0, 0)),
            pl.BlockSpec((1, 2, 16, C), lambda i, e: (e[0], 0, 0, 0)),
        ],
        out_specs=[
            pl.BlockSpec((B, C), lambda i, e: (i, 0)),
            pl.BlockSpec((1, 1, 1), lambda i, e: (i, 0, 0),
                         memory_space=pltpu.SMEM),
        ],
    )
    feats, bits = pl.pallas_call(
        _forward_kernel,
        grid_spec=grid_spec,
        out_shape=[
            jax.ShapeDtypeStruct((N, C), jnp.float32),
            jax.ShapeDtypeStruct((NB, 1, 1), jnp.float32),
        ],
        compiler_params=pltpu.CompilerParams(
            dimension_semantics=("parallel",)),
    )(enc1, x_O.reshape(NB, B, 1), feats_prop, p['blend_w'], p['local_W1'],
      lb1, p['local_W2'], lb2, p['res_W1'], p['res_W2'], p['pred_W1'],
      p['pred_b1'], p['pred_W2'], p['pred_b2'], p['prior_emb'])
    return jnp.sum(bits) / N, feats


# speedup vs baseline: 1.1473x; 1.1473x over previous
import jax
import jax.numpy as jnp
from jax import lax
from jax.experimental import pallas as pl
from jax.experimental.pallas import tpu as pltpu

N = 50000
C = 128
K = 4
E = K + 1
B = 5000
NB = N // B


def _routing_kernel(xo_ref, xc_ref, cen_ref, enc_ref):
    xo = xo_ref[...]
    s0 = xo & 15
    s1 = xo >> 4
    cnt = [jnp.sum((s == j).astype(jnp.float32)) for s in (s0, s1)
           for j in range(16)]
    tot = cnt[0]
    for c in cnt[1:]:
        tot = tot + c
    xh = [c / tot for c in cnt]
    d2 = []
    for k in range(K):
        acc = (cen_ref[k, 0] - xh[0]) ** 2
        for j in range(1, 32):
            acc = acc + (cen_ref[k, j] - xh[j]) ** 2
        d2.append(acc)
    idx = jnp.int32(0)
    best = d2[0]
    for k in range(1, K):
        pred = d2[k] < best
        idx = jnp.where(pred, jnp.int32(k), idx)
        best = jnp.where(pred, d2[k], best)
    xc = xc_ref[...]
    lane = lax.broadcasted_iota(jnp.int32, xc.shape, 1)
    mc = jnp.max(jnp.where(lane % 4 != 0, xc, 0))
    enc_ref[0, 0] = jnp.where(mc <= 63, jnp.int32(K), idx)


def _forward_kernel(e_ref, xo_ref, fp_ref, bw_ref, lw1_ref, lb1_ref, lw2_ref,
                    lb2_ref, rw1_ref, rw2_ref, pw1_ref, pb1_ref, pw2_ref,
                    pb2_ref, pe_ref, out_ref, bits_ref):
    del e_ref
    i = pl.program_id(0)
    xo = xo_ref[0, :, :]
    iota16 = lax.broadcasted_iota(jnp.int32, (B, 16), 1)
    oh0 = ((xo & 15) == iota16).astype(jnp.float32)
    oh1 = ((xo >> 4) == iota16).astype(jnp.float32)

    def mm(a, b):
        return jnp.dot(a, b, preferred_element_type=jnp.float32)

    def catn(a, b):
        return jnp.concatenate([a, b], axis=1)

    def catk(a, b):
        return jnp.concatenate([a, b], axis=0)

    h = jnp.maximum(lw1_ref[0] + lb1_ref[0], 0.0)
    row = mm(h, lw2_ref[0]) + lb2_ref[0]

    bw = bw_ref[0]
    bmx = jnp.max(bw, axis=0, keepdims=True)
    be = jnp.exp(bw - bmx)
    wsm = be / jnp.sum(be, axis=0, keepdims=True)
    f = wsm[0:1, :] * row + wsm[1:2, :] * fp_ref[...]

    def res_pair(f, ta, ja, jb):
        ab = rw1_ref[0, jb]
        tb = jnp.maximum(mm(catn(f, ta),
                            catk(ab, mm(rw2_ref[0, ja], ab))), 0.0)
        return f + mm(catn(ta, tb), catk(rw2_ref[0, ja], rw2_ref[0, jb]))

    def head(f, t, oh, jnext):
        anext = rw1_ref[0, jnext]
        hp_u = mm(f, catn(pw1_ref[0, t], anext))
        emb_v = mm(oh, catn(pe_ref[0, t], mm(pe_ref[0, t], anext)))
        hh = jnp.maximum(hp_u[:, :C] + pb1_ref[0, t], 0.0)
        lg = mm(hh, pw2_ref[0, t]) + pb2_ref[0, t]
        mx = jnp.max(lg, axis=1, keepdims=True)
        ex = jnp.exp(lg - mx)
        z = jnp.sum(ex, axis=1, keepdims=True)
        exs = jnp.sum(ex * oh, axis=1, keepdims=True)
        ps = exs / z
        bits = jnp.sum(jnp.clip(-jnp.log2(ps + 1e-10), 0.0, 50.0))
        tnext = jnp.maximum(hp_u[:, C:] + emb_v[:, C:], 0.0)
        return bits, f + emb_v[:, :C], tnext

    t0 = jnp.maximum(mm(f, rw1_ref[0, 0]), 0.0)
    f = res_pair(f, t0, 0, 1)
    bits_a, f, t2 = head(f, 0, oh0, 2)
    f = res_pair(f, t2, 2, 3)
    bits_b, f, t4 = head(f, 1, oh1, 4)
    f = res_pair(f, t4, 4, 5)
    out_ref[...] = f
    bits_ref[0, 0, 0] = bits_a + bits_b


def kernel(x_C, x_O, feats_prop, centers, params):
    xo2 = x_O.reshape(400, 125)
    xc2 = x_C.reshape(1000, 200)
    enc = pl.pallas_call(
        _routing_kernel,
        out_shape=jax.ShapeDtypeStruct((1, 1), jnp.int32),
        in_specs=[
            pl.BlockSpec(memory_space=pltpu.VMEM),
            pl.BlockSpec(memory_space=pltpu.VMEM),
            pl.BlockSpec(memory_space=pltpu.SMEM),
        ],
        out_specs=pl.BlockSpec(memory_space=pltpu.SMEM),
    )(xo2, xc2, centers)
    enc1 = enc.reshape((1,))

    p = params
    lb1 = p['local_b1'][:, None, :]
    lb2 = p['local_b2'][:, None, :]

    grid_spec = pltpu.PrefetchScalarGridSpec(
        num_scalar_prefetch=1,
        grid=(NB,),
        in_specs=[
            pl.BlockSpec((1, B, 1), lambda i, e: (i, 0, 0)),
            pl.BlockSpec((B, C), lambda i, e: (i, 0)),
            pl.BlockSpec((1, 2, C), lambda i, e: (e[0], 0, 0)),
            pl.BlockSpec((1, 1, C), lambda i, e: (e[0], 0, 0)),
            pl.BlockSpec((1, 1, C), lambda i, e: (e[0], 0, 0)),
            pl.BlockSpec((1, C, C), lambda i, e: (e[0], 0, 0)),
            pl.BlockSpec((1, 1, C), lambda i, e: (e[0], 0, 0)),
            pl.BlockSpec((1, 6, C, C), lambda i, e: (e[0], 0, 0, 0)),
            pl.BlockSpec((1, 6, C, C), lambda i, e: (e[0], 0, 0, 0)),
            pl.BlockSpec((1, 2, C, C), lambda i, e: (e[0], 0, 0, 0)),
            pl.BlockSpec((1, 2, C), lambda i, e: (e[0], 0, 0)),
            pl.BlockSpec((1, 2, C, 16), lambda i, e: (e[0], 0, 0, 0)),
            pl.BlockSpec((1, 2, 16), lambda i, e: (e[0], 0, 0)),
            pl.BlockSpec((1, 2, 16, C), lambda i, e: (e[0], 0, 0, 0)),
        ],
        out_specs=[
            pl.BlockSpec((B, C), lambda i, e: (i, 0)),
            pl.BlockSpec((1, 1, 1), lambda i, e: (i, 0, 0),
                         memory_space=pltpu.SMEM),
        ],
    )
    feats, bits = pl.pallas_call(
        _forward_kernel,
        grid_spec=grid_spec,
        out_shape=[
            jax.ShapeDtypeStruct((N, C), jnp.float32),
            jax.ShapeDtypeStruct((NB, 1, 1), jnp.float32),
        ],
        compiler_params=pltpu.CompilerParams(
            dimension_semantics=("parallel",)),
    )(enc1, x_O.reshape(NB, B, 1), feats_prop, p['blend_w'], p['local_W1'],
      lb1, p['local_W2'], lb2, p['res_W1'], p['res_W2'], p['pred_W1'],
      p['pred_b1'], p['pred_W2'], p['pred_b2'], p['prior_emb'])
    return jnp.sum(bits) / N, feats


# speedup vs baseline: 1.1647x; 1.0151x over previous
import jax
import jax.numpy as jnp
from jax import lax
from jax.experimental import pallas as pl
from jax.experimental.pallas import tpu as pltpu

N = 50000
C = 128
K = 4
E = K + 1
B = 5000
NB = N // B


def _kernel(xof_ref, xcf_ref, cen_ref, xo_ref, fp_ref, bw_ref, lw1_ref,
            lb1_ref, lw2_ref, lb2_ref, rw1_ref, rw2_ref, pw1_ref, pb1_ref,
            pw2_ref, pb2_ref, pe_ref, out_ref, bits_ref, enc_ref):
    i = pl.program_id(0)

    @pl.when(i == 0)
    def _():
        xof = xof_ref[...]
        v0 = xof & 15
        v1 = xof >> 4
        cnt = [jnp.sum((s == j).astype(jnp.float32)) for s in (v0, v1)
               for j in range(16)]
        tot = cnt[0]
        for c in cnt[1:]:
            tot = tot + c
        xh = [c / tot for c in cnt]
        d2 = []
        for k in range(K):
            acc = (cen_ref[k, 0] - xh[0]) ** 2
            for j in range(1, 32):
                acc = acc + (cen_ref[k, j] - xh[j]) ** 2
            d2.append(acc)
        idx = jnp.int32(0)
        best = d2[0]
        for k in range(1, K):
            pred = d2[k] < best
            idx = jnp.where(pred, jnp.int32(k), idx)
            best = jnp.where(pred, d2[k], best)
        xc = xcf_ref[...]
        lanes = lax.broadcasted_iota(jnp.int32, xc.shape, 1)
        mc = jnp.max(jnp.where(lanes % 4 != 0, xc, 0))
        enc_ref[0] = jnp.where(mc <= 63, jnp.int32(K), idx)
        bits_ref[0, 0, 0] = 0.0

    e = enc_ref[0]
    xo = xo_ref[0, :, :]
    iota16 = lax.broadcasted_iota(jnp.int32, (B, 16), 1)
    oh0 = ((xo & 15) == iota16).astype(jnp.float32)
    oh1 = ((xo >> 4) == iota16).astype(jnp.float32)

    def mm(a, b):
        return jnp.dot(a, b, preferred_element_type=jnp.float32)

    def catn(a, b):
        return jnp.concatenate([a, b], axis=1)

    def catk(a, b):
        return jnp.concatenate([a, b], axis=0)

    h = jnp.maximum(lw1_ref[e, 0] + lb1_ref[e, 0], 0.0)[None, :]
    row = mm(h, lw2_ref[e]) + lb2_ref[e, 0][None, :]

    bw = bw_ref[e]
    bmx = jnp.max(bw, axis=0, keepdims=True)
    be = jnp.exp(bw - bmx)
    wsm = be / jnp.sum(be, axis=0, keepdims=True)
    f = wsm[0:1, :] * row + wsm[1:2, :] * fp_ref[...]

    def res_pair(f, ta, ja, jb):
        ab = rw1_ref[e, jb]
        tb = jnp.maximum(mm(catn(f, ta),
                            catk(ab, mm(rw2_ref[e, ja], ab))), 0.0)
        return f + mm(catn(ta, tb), catk(rw2_ref[e, ja], rw2_ref[e, jb]))

    def head(f, t, oh, jnext):
        anext = rw1_ref[e, jnext]
        hp_u = mm(f, catn(pw1_ref[e, t], anext))
        emb_v = mm(oh, catn(pe_ref[e, t], mm(pe_ref[e, t], anext)))
        hh = jnp.maximum(hp_u[:, :C] + pb1_ref[e, t][None, :], 0.0)
        lg = mm(hh, pw2_ref[e, t]) + pb2_ref[e, t][None, :]
        mx = jnp.max(lg, axis=1, keepdims=True)
        ex = jnp.exp(lg - mx)
        z = jnp.sum(ex, axis=1, keepdims=True)
        exs = jnp.sum(ex * oh, axis=1, keepdims=True)
        ps = exs / z
        bits = jnp.sum(jnp.clip(-jnp.log2(ps + 1e-10), 0.0, 50.0))
        tnext = jnp.maximum(hp_u[:, C:] + emb_v[:, C:], 0.0)
        return bits, f + emb_v[:, :C], tnext

    t0 = jnp.maximum(mm(f, rw1_ref[e, 0]), 0.0)
    f = res_pair(f, t0, 0, 1)
    bits_a, f, t2 = head(f, 0, oh0, 2)
    f = res_pair(f, t2, 2, 3)
    bits_b, f, t4 = head(f, 1, oh1, 4)
    f = res_pair(f, t4, 4, 5)
    out_ref[...] = f

    bits_ref[0, 0, 0] = bits_ref[0, 0, 0] + (bits_a + bits_b)

    @pl.when(i == NB - 1)
    def _():
        bits_ref[0, 0, 0] = bits_ref[0, 0, 0] / N


def kernel(x_C, x_O, feats_prop, centers, params):
    p = params
    lb1 = p['local_b1'][:, None, :]
    lb2 = p['local_b2'][:, None, :]

    full = pl.BlockSpec(memory_space=pltpu.VMEM)
    feats, bits = pl.pallas_call(
        _kernel,
        grid=(NB,),
        in_specs=[
            full,
            full,
            pl.BlockSpec(memory_space=pltpu.SMEM),
            pl.BlockSpec((1, B, 1), lambda i: (i, 0, 0)),
            pl.BlockSpec((B, C), lambda i: (i, 0)),
            full, full, full, full, full,
            full, full, full, full, full, full, full,
        ],
        out_specs=[
            pl.BlockSpec((B, C), lambda i: (i, 0)),
            pl.BlockSpec((1, 1, 1), lambda i: (0, 0, 0),
                         memory_space=pltpu.SMEM),
        ],
        out_shape=[
            jax.ShapeDtypeStruct((N, C), jnp.float32),
            jax.ShapeDtypeStruct((1, 1, 1), jnp.float32),
        ],
        scratch_shapes=[pltpu.SMEM((1,), jnp.int32)],
        compiler_params=pltpu.CompilerParams(
            dimension_semantics=("arbitrary",)),
    )(x_O.reshape(400, 125), x_C.reshape(1000, 200), centers,
      x_O.reshape(NB, B, 1), feats_prop, p['blend_w'], p['local_W1'], lb1,
      p['local_W2'], lb2, p['res_W1'], p['res_W2'], p['pred_W1'],
      p['pred_b1'], p['pred_W2'], p['pred_b2'], p['prior_emb'])
    return bits[0, 0, 0], feats


# speedup vs baseline: 1.5152x; 1.3010x over previous
import jax
import jax.numpy as jnp
from jax import lax
from jax.experimental import pallas as pl
from jax.experimental.pallas import tpu as pltpu

N = 50000
C = 128
K = 4
E = K + 1
B = 10000
NB = N // B


def _kernel(xof_ref, xcf_ref, cen_ref, xo_ref, fp_ref, bw_ref, lw1_ref,
            lb1_ref, lw2_ref, lb2_ref, rw1_ref, rw2_ref, pw1_ref, pb1_ref,
            pw2_ref, pb2_ref, pe_ref, out_ref, bits_ref, enc_ref):
    i = pl.program_id(0)

    @pl.when(i == 0)
    def _():
        xof = xof_ref[...]
        v0 = xof & 15
        v1 = xof >> 4
        cnt = [jnp.sum((s == j).astype(jnp.float32)) for s in (v0, v1)
               for j in range(16)]
        tot = cnt[0]
        for c in cnt[1:]:
            tot = tot + c
        xh = [c / tot for c in cnt]
        d2 = []
        for k in range(K):
            acc = (cen_ref[k, 0] - xh[0]) ** 2
            for j in range(1, 32):
                acc = acc + (cen_ref[k, j] - xh[j]) ** 2
            d2.append(acc)
        idx = jnp.int32(0)
        best = d2[0]
        for k in range(1, K):
            pred = d2[k] < best
            idx = jnp.where(pred, jnp.int32(k), idx)
            best = jnp.where(pred, d2[k], best)
        xc = xcf_ref[...]
        lanes = lax.broadcasted_iota(jnp.int32, xc.shape, 1)
        mc = jnp.max(jnp.where(lanes % 4 != 0, xc, 0))
        enc_ref[0] = jnp.where(mc <= 63, jnp.int32(K), idx)
        bits_ref[0, 0, 0] = 0.0

    e = enc_ref[0]
    xo = xo_ref[0, 0, :][:, None]
    iota16 = lax.broadcasted_iota(jnp.int32, (B, 16), 1)
    oh0 = ((xo & 15) == iota16).astype(jnp.float32)
    oh1 = ((xo >> 4) == iota16).astype(jnp.float32)

    def mm(a, b):
        return jnp.dot(a, b, preferred_element_type=jnp.float32)

    def catn(a, b):
        return jnp.concatenate([a, b], axis=1)

    def catk(a, b):
        return jnp.concatenate([a, b], axis=0)

    h = jnp.maximum(lw1_ref[e, 0] + lb1_ref[e, 0], 0.0)[None, :]
    row = mm(h, lw2_ref[e]) + lb2_ref[e, 0][None, :]

    bw = bw_ref[e]
    bmx = jnp.max(bw, axis=0, keepdims=True)
    be = jnp.exp(bw - bmx)
    wsm = be / jnp.sum(be, axis=0, keepdims=True)
    f = wsm[0:1, :] * row + wsm[1:2, :] * fp_ref[...]

    def res_pair(f, ta, ja, jb):
        ab = rw1_ref[e, jb]
        tb = jnp.maximum(mm(catn(f, ta),
                            catk(ab, mm(rw2_ref[e, ja], ab))), 0.0)
        return f + mm(catn(ta, tb), catk(rw2_ref[e, ja], rw2_ref[e, jb]))

    def head(f, t, oh, jnext):
        anext = rw1_ref[e, jnext]
        hp_u = mm(f, catn(pw1_ref[e, t], anext))
        emb_v = mm(oh, catn(pe_ref[e, t], mm(pe_ref[e, t], anext)))
        hh = jnp.maximum(hp_u[:, :C] + pb1_ref[e, t][None, :], 0.0)
        lg = mm(hh, pw2_ref[e, t]) + pb2_ref[e, t][None, :]
        mx = jnp.max(lg, axis=1, keepdims=True)
        ex = jnp.exp(lg - mx)
        z = jnp.sum(ex, axis=1, keepdims=True)
        exs = jnp.sum(ex * oh, axis=1, keepdims=True)
        ps = exs / z
        bits = jnp.sum(jnp.clip(-jnp.log2(ps + 1e-10), 0.0, 50.0))
        tnext = jnp.maximum(hp_u[:, C:] + emb_v[:, C:], 0.0)
        return bits, f + emb_v[:, :C], tnext

    t0 = jnp.maximum(mm(f, rw1_ref[e, 0]), 0.0)
    f = res_pair(f, t0, 0, 1)
    bits_a, f, t2 = head(f, 0, oh0, 2)
    f = res_pair(f, t2, 2, 3)
    bits_b, f, t4 = head(f, 1, oh1, 4)
    f = res_pair(f, t4, 4, 5)
    out_ref[...] = f

    bits_ref[0, 0, 0] = bits_ref[0, 0, 0] + (bits_a + bits_b)

    @pl.when(i == NB - 1)
    def _():
        bits_ref[0, 0, 0] = bits_ref[0, 0, 0] / N


def kernel(x_C, x_O, feats_prop, centers, params):
    p = params
    lb1 = p['local_b1'][:, None, :]
    lb2 = p['local_b2'][:, None, :]

    full = pl.BlockSpec(memory_space=pltpu.VMEM)
    feats, bits = pl.pallas_call(
        _kernel,
        grid=(NB,),
        in_specs=[
            full,
            full,
            pl.BlockSpec(memory_space=pltpu.SMEM),
            pl.BlockSpec((1, 1, B), lambda i: (i, 0, 0)),
            pl.BlockSpec((B, C), lambda i: (i, 0)),
            full, full, full, full, full,
            full, full, full, full, full, full, full,
        ],
        out_specs=[
            pl.BlockSpec((B, C), lambda i: (i, 0)),
            pl.BlockSpec((1, 1, 1), lambda i: (0, 0, 0),
                         memory_space=pltpu.SMEM),
        ],
        out_shape=[
            jax.ShapeDtypeStruct((N, C), jnp.float32),
            jax.ShapeDtypeStruct((1, 1, 1), jnp.float32),
        ],
        scratch_shapes=[pltpu.SMEM((1,), jnp.int32)],
        compiler_params=pltpu.CompilerParams(
            dimension_semantics=("arbitrary",)),
    )(x_O.reshape(400, 125), x_C.reshape(1000, 200), centers,
      x_O.reshape(NB, 1, B), feats_prop, p['blend_w'], p['local_W1'], lb1,
      p['local_W2'], lb2, p['res_W1'], p['res_W2'], p['pred_W1'],
      p['pred_b1'], p['pred_W2'], p['pred_b2'], p['prior_emb'])
    return bits[0, 0, 0], feats
